# Initial kernel scaffold; baseline (speedup 1.0000x reference)
import functools

import jax
import jax.numpy as jnp
from jax import lax
from jax.experimental import pallas as pl
from jax.experimental.pallas import tpu as pltpu
from jax.experimental.pallas import tpu_sc as plsc

N = 10000
D = 128
E = 320000
NC = 2
NS = 16
NW = NC * NS
EPW = E // NW
CHUNK = 80
NCH = EPW // CHUNK
RPT = N // NS
WB = 125
NWB = RPT // WB
NPAD = 10240
DPT = NPAD // NS

_mesh = plsc.VectorSubcoreMesh(core_axis_name="c", subcore_axis_name="s")


@functools.partial(
    pl.kernel,
    out_type=jax.ShapeDtypeStruct((NC, NPAD), jnp.float32),
    mesh=_mesh,
    scratch_types=[
        pltpu.VMEM((NCH, CHUNK), jnp.int32),
        pltpu.VMEM((CHUNK,), jnp.float32),
        pltpu.VMEM((DPT,), jnp.float32),
        pltpu.VMEM_SHARED((NPAD,), jnp.float32),
    ],
)
def _deg_kernel(dst_hbm, out_hbm, dstv, ones, wb, acc):
    c = lax.axis_index("c")
    s = lax.axis_index("s")
    w = s * NC + c

    @pl.loop(0, DPT // 16)
    def _zero(i):
        wb[pl.ds(i * 16, 16)] = jnp.zeros((16,), jnp.float32)

    @pl.loop(0, CHUNK // 16)
    def _one(i):
        ones[pl.ds(i * 16, 16)] = jnp.ones((16,), jnp.float32)

    pltpu.sync_copy(wb, acc.at[pl.ds(s * DPT, DPT)])
    plsc.subcore_barrier()

    pltpu.sync_copy(dst_hbm.at[w], dstv)

    @pl.loop(0, NCH)
    def _edges(ch):
        pltpu.sync_copy(ones, acc.at[dstv.at[ch]], add=True)

    plsc.subcore_barrier()
    pltpu.sync_copy(acc.at[pl.ds(s * DPT, DPT)], wb)
    pltpu.sync_copy(wb, out_hbm.at[c, pl.ds(s * DPT, DPT)])


@functools.partial(
    pl.kernel,
    out_type=jax.ShapeDtypeStruct((NC, N, D), jnp.float32),
    mesh=_mesh,
    scratch_types=[
        pltpu.VMEM((NCH, CHUNK), jnp.int32),
        pltpu.VMEM((NCH, CHUNK), jnp.int32),
        pltpu.VMEM((CHUNK, D), jnp.float32),
        pltpu.VMEM((WB, D), jnp.float32),
        pltpu.VMEM_SHARED((N, D), jnp.float32),
        pltpu.SemaphoreType.DMA,
    ],
)
def _spmm_kernel(xp_hbm, src_hbm, dst_hbm, out_hbm, srcv, dstv, rows, wb, acc, sem):
    c = lax.axis_index("c")
    s = lax.axis_index("s")
    w = s * NC + c

    @pl.loop(0, WB)
    def _zero(r):
        for j in range(D // 16):
            wb[r, pl.ds(j * 16, 16)] = jnp.zeros((16,), jnp.float32)

    for j in range(NWB):
        pltpu.sync_copy(wb, acc.at[pl.ds(s * RPT + j * WB, WB), :])
    plsc.subcore_barrier()

    pltpu.sync_copy(src_hbm.at[w], srcv)
    pltpu.sync_copy(dst_hbm.at[w], dstv)

    @pl.loop(0, NCH)
    def _edges(ch):
        pltpu.async_copy(xp_hbm.at[srcv.at[ch]], rows, sem).wait()
        pltpu.sync_copy(rows, acc.at[dstv.at[ch]], add=True)

    plsc.subcore_barrier()
    for j in range(NWB):
        base = s * RPT + j * WB
        pltpu.sync_copy(acc.at[pl.ds(base, WB), :], wb)
        pltpu.sync_copy(wb, out_hbm.at[c, pl.ds(base, WB), :])


def _scale_body(x_ref, degp_ref, xp_ref, dinv_ref):
    deg = degp_ref[:, 0:1] + degp_ref[:, 1:2] + 1.0
    dinv = lax.rsqrt(deg)
    dinv_ref[...] = dinv
    xp_ref[...] = x_ref[...] * dinv


_scale_call = pl.pallas_call(
    _scale_body,
    out_shape=(
        jax.ShapeDtypeStruct((N, D), jnp.float32),
        jax.ShapeDtypeStruct((N, 1), jnp.float32),
    ),
)


def _dense_body(p_ref, xp_ref, dinv_ref, w1_ref, b1_ref, w2_ref, tp_ref):
    dinv = dinv_ref[...]
    s1 = (p_ref[0] + p_ref[1] + xp_ref[...]) * dinv
    h = jnp.dot(s1, w1_ref[...], preferred_element_type=jnp.float32)
    h = jnp.maximum(h + b1_ref[...].reshape(1, -1), 0.0)
    t = jnp.dot(h, w2_ref[...], preferred_element_type=jnp.float32)
    tp_ref[...] = t * dinv


_dense_call = pl.pallas_call(
    _dense_body,
    out_shape=jax.ShapeDtypeStruct((N, D), jnp.float32),
)


def _softmax_body(q_ref, tp_ref, dinv_ref, b2_ref, o_ref):
    s2 = (q_ref[0] + q_ref[1] + tp_ref[...]) * dinv_ref[...]
    s2 = s2 + b2_ref[...].reshape(1, -1)
    m = jnp.max(s2, axis=1, keepdims=True)
    e = jnp.exp(s2 - m)
    o_ref[...] = e / jnp.sum(e, axis=1, keepdims=True)


_softmax_call = pl.pallas_call(
    _softmax_body,
    out_shape=jax.ShapeDtypeStruct((N, D), jnp.float32),
)


def kernel(x, edge_index, W1, b1, W2, b2):
    ei = edge_index.astype(jnp.int32)
    src = ei[0].reshape(NW, NCH, CHUNK)
    dst = ei[1].reshape(NW, NCH, CHUNK)

    deg_p = _deg_kernel(dst)
    deg_p = deg_p[:, :N].T
    xp, dinv = _scale_call(x, deg_p)
    p = _spmm_kernel(xp, src, dst)
    tp = _dense_call(p, xp, dinv, W1, b1, W2)
    q = _spmm_kernel(tp, src, dst)
    return _softmax_call(q, tp, dinv, b2)


# speedup vs baseline: 22.0454x; 22.0454x over previous
import functools

import jax
import jax.numpy as jnp
from jax import lax
from jax.experimental import pallas as pl
from jax.experimental.pallas import tpu as pltpu
from jax.experimental.pallas import tpu_sc as plsc

N = 10000
D = 128
E = 320000
NC = 2
NS = 16
NW = NC * NS
EPW = E // NW
CHUNK = 80
NCH = EPW // CHUNK
SCH = 25
NSB = NCH // SCH
NPAD = 10240
RPT = NPAD // NS
WB = 64
NWB = RPT // WB
DPT = NPAD // NS

_mesh = plsc.VectorSubcoreMesh(core_axis_name="c", subcore_axis_name="s")


@functools.partial(
    pl.kernel,
    out_type=jax.ShapeDtypeStruct((NC * NPAD,), jnp.float32),
    mesh=_mesh,
    scratch_types=[
        pltpu.VMEM((SCH, CHUNK), jnp.int32),
        pltpu.VMEM((CHUNK,), jnp.float32),
        pltpu.VMEM((DPT,), jnp.float32),
        pltpu.VMEM_SHARED((NPAD,), jnp.float32),
    ],
)
def _deg_kernel(dst_hbm, out_hbm, dstv, ones, wb, acc):
    c = lax.axis_index("c")
    s = lax.axis_index("s")
    w = s * NC + c

    @pl.loop(0, DPT // 16)
    def _zero(i):
        wb[pl.ds(i * 16, 16)] = jnp.zeros((16,), jnp.float32)

    @pl.loop(0, CHUNK // 16)
    def _one(i):
        ones[pl.ds(i * 16, 16)] = jnp.ones((16,), jnp.float32)

    pltpu.sync_copy(wb, acc.at[pl.ds(s * DPT, DPT)])
    plsc.subcore_barrier()

    @pl.loop(0, NSB)
    def _blocks(bk):
        pltpu.sync_copy(dst_hbm.at[w, bk], dstv)

        @pl.loop(0, SCH)
        def _edges(ch):
            pltpu.sync_copy(ones, acc.at[dstv.at[ch]], add=True)

    plsc.subcore_barrier()
    pltpu.sync_copy(acc.at[pl.ds(s * DPT, DPT)], wb)
    pltpu.sync_copy(wb, out_hbm.at[pl.ds(c * NPAD + s * DPT, DPT)])


@functools.partial(
    pl.kernel,
    out_type=jax.ShapeDtypeStruct((NC, NPAD, D), jnp.float32),
    mesh=_mesh,
    scratch_types=[
        pltpu.VMEM((SCH, CHUNK), jnp.int32),
        pltpu.VMEM((SCH, CHUNK), jnp.int32),
        pltpu.VMEM((CHUNK, D), jnp.float32),
        pltpu.VMEM((WB, D), jnp.float32),
        pltpu.VMEM_SHARED((NPAD, D), jnp.float32),
        pltpu.SemaphoreType.DMA,
    ],
)
def _spmm_kernel(xp_hbm, src_hbm, dst_hbm, out_hbm, srcv, dstv, rows, wb, acc, sem):
    c = lax.axis_index("c")
    s = lax.axis_index("s")
    w = s * NC + c

    @pl.loop(0, WB)
    def _zero(r):
        for j in range(D // 16):
            wb[r, pl.ds(j * 16, 16)] = jnp.zeros((16,), jnp.float32)

    for j in range(NWB):
        pltpu.sync_copy(wb, acc.at[pl.ds(s * RPT + j * WB, WB), :])
    plsc.subcore_barrier()

    @pl.loop(0, NSB)
    def _blocks(bk):
        pltpu.sync_copy(src_hbm.at[w, bk], srcv)
        pltpu.sync_copy(dst_hbm.at[w, bk], dstv)

        @pl.loop(0, SCH)
        def _edges(ch):
            pltpu.async_copy(xp_hbm.at[srcv.at[ch]], rows, sem).wait()
            pltpu.sync_copy(rows, acc.at[dstv.at[ch]], add=True)

    plsc.subcore_barrier()
    for j in range(NWB):
        base = s * RPT + j * WB
        pltpu.sync_copy(acc.at[pl.ds(base, WB), :], wb)
        pltpu.sync_copy(wb, out_hbm.at[c, pl.ds(base, WB), :])


def _scale_body(x_ref, degp_ref, xp_ref, dinv_ref):
    deg = degp_ref[:, 0:1] + degp_ref[:, 1:2] + 1.0
    dinv = lax.rsqrt(deg)
    dinv_ref[...] = dinv
    xp_ref[...] = x_ref[...] * dinv


_scale_call = pl.pallas_call(
    _scale_body,
    out_shape=(
        jax.ShapeDtypeStruct((N, D), jnp.float32),
        jax.ShapeDtypeStruct((N, 1), jnp.float32),
    ),
)


def _dense_body(p_ref, xp_ref, dinv_ref, w1_ref, b1_ref, w2_ref, tp_ref):
    dinv = dinv_ref[...]
    s1 = (p_ref[0, :N] + p_ref[1, :N] + xp_ref[...]) * dinv
    h = jnp.dot(s1, w1_ref[...], preferred_element_type=jnp.float32)
    h = jnp.maximum(h + b1_ref[...].reshape(1, -1), 0.0)
    t = jnp.dot(h, w2_ref[...], preferred_element_type=jnp.float32)
    tp_ref[...] = t * dinv


_dense_call = pl.pallas_call(
    _dense_body,
    out_shape=jax.ShapeDtypeStruct((N, D), jnp.float32),
)


def _softmax_body(q_ref, tp_ref, dinv_ref, b2_ref, o_ref):
    s2 = (q_ref[0, :N] + q_ref[1, :N] + tp_ref[...]) * dinv_ref[...]
    s2 = s2 + b2_ref[...].reshape(1, -1)
    m = jnp.max(s2, axis=1, keepdims=True)
    e = jnp.exp(s2 - m)
    o_ref[...] = e / jnp.sum(e, axis=1, keepdims=True)


_softmax_call = pl.pallas_call(
    _softmax_body,
    out_shape=jax.ShapeDtypeStruct((N, D), jnp.float32),
)


def kernel(x, edge_index, W1, b1, W2, b2):
    ei = edge_index.astype(jnp.int32)
    src = ei[0].reshape(NW, NSB, SCH, CHUNK)
    dst = ei[1].reshape(NW, NSB, SCH, CHUNK)

    deg_p = _deg_kernel(dst).reshape(NC, NPAD)
    deg_p = deg_p[:, :N].T
    xp, dinv = _scale_call(x, deg_p)
    p = _spmm_kernel(xp, src, dst)
    tp = _dense_call(p, xp, dinv, W1, b1, W2)
    q = _spmm_kernel(tp, src, dst)
    return _softmax_call(q, tp, dinv, b2)
